# baseline (speedup 1.0000x reference)
import functools

import jax
import jax.numpy as jnp
from jax import lax
from jax.experimental import pallas as pl
from jax.experimental.pallas import tpu as pltpu
from jax.experimental.pallas import tpu_sc as plsc

_NC, _NS = 2, 16
_NW = _NC * _NS
_CHUNK = 800


@functools.cache
def _build_gather(n, d):
    n_per_w = n // _NW
    n_chunks = n_per_w // _CHUNK
    mesh = plsc.VectorSubcoreMesh(core_axis_name="c", subcore_axis_name="s")

    @functools.partial(
        pl.kernel,
        out_type=jax.ShapeDtypeStruct((n, d), jnp.float32),
        mesh=mesh,
        scratch_types=[
            pltpu.VMEM((_CHUNK,), jnp.int32),
            pltpu.VMEM((_CHUNK, d), jnp.float32),
            pltpu.SemaphoreType.DMA,
        ],
        compiler_params=pltpu.CompilerParams(use_tc_tiling_on_sc=False),
    )
    def gather(idx_hbm, table_hbm, out_hbm, idx_v, rows_v, sem):
        wid = lax.axis_index("s") * _NC + lax.axis_index("c")
        base = wid * n_per_w

        @pl.loop(0, n_chunks)
        def _chunk(i):
            off = base + i * _CHUNK
            pltpu.sync_copy(idx_hbm.at[pl.ds(off, _CHUNK)], idx_v)
            pltpu.async_copy(table_hbm.at[idx_v], rows_v, sem).wait()
            pltpu.sync_copy(rows_v, out_hbm.at[pl.ds(off, _CHUNK)])

    return gather


def kernel(sequences_tokens, embedding_table):
    b, l = sequences_tokens.shape
    _, d = embedding_table.shape
    idx = sequences_tokens.reshape(b * l)
    out = _build_gather(b * l, d)(idx, embedding_table)
    return out.reshape(b, l, d)


# speedup vs baseline: 1.0272x; 1.0272x over previous
import functools

import jax
import jax.numpy as jnp
from jax import lax
from jax.experimental import pallas as pl
from jax.experimental.pallas import tpu as pltpu
from jax.experimental.pallas import tpu_sc as plsc

_NC, _NS = 2, 16
_NW = _NC * _NS
_CHUNK = 800


@functools.cache
def _build_gather(n, d):
    n_per_w = n // _NW
    n_chunks = n_per_w // _CHUNK
    assert n_chunks % 2 == 0 and n_chunks >= 4
    mesh = plsc.VectorSubcoreMesh(core_axis_name="c", subcore_axis_name="s")

    @functools.partial(
        pl.kernel,
        out_type=jax.ShapeDtypeStruct((n, d), jnp.float32),
        mesh=mesh,
        scratch_types=[
            pltpu.VMEM((_CHUNK,), jnp.int32),
            pltpu.VMEM((_CHUNK,), jnp.int32),
            pltpu.VMEM((_CHUNK, d), jnp.float32),
            pltpu.VMEM((_CHUNK, d), jnp.float32),
            pltpu.SemaphoreType.DMA,
            pltpu.SemaphoreType.DMA,
            pltpu.SemaphoreType.DMA,
            pltpu.SemaphoreType.DMA,
            pltpu.SemaphoreType.DMA,
            pltpu.SemaphoreType.DMA,
        ],
        compiler_params=pltpu.CompilerParams(use_tc_tiling_on_sc=False),
    )
    def gather(idx_hbm, table_hbm, out_hbm,
               idx0, idx1, rows0, rows1,
               isem0, isem1, gsem0, gsem1, wsem0, wsem1):
        wid = lax.axis_index("s") * _NC + lax.axis_index("c")
        base = wid * n_per_w
        idx_v = (idx0, idx1)
        rows_v = (rows0, rows1)
        isem = (isem0, isem1)
        gsem = (gsem0, gsem1)
        wsem = (wsem0, wsem1)

        def idx_start(i, b):
            pltpu.async_copy(
                idx_hbm.at[pl.ds(base + i * _CHUNK, _CHUNK)], idx_v[b], isem[b])

        def idx_wait(b):
            pltpu.make_async_copy(
                idx_hbm.at[pl.ds(base, _CHUNK)], idx_v[b], isem[b]).wait()

        def gather_start(b):
            pltpu.async_copy(table_hbm.at[idx_v[b]], rows_v[b], gsem[b])

        def gather_wait(b):
            pltpu.make_async_copy(
                table_hbm.at[idx_v[b]], rows_v[b], gsem[b]).wait()

        def write_start(i, b):
            pltpu.async_copy(
                rows_v[b], out_hbm.at[pl.ds(base + i * _CHUNK, _CHUNK)], wsem[b])

        def write_wait(b):
            pltpu.make_async_copy(
                rows_v[b], out_hbm.at[pl.ds(base, _CHUNK)], wsem[b]).wait()

        idx_start(0, 0)
        idx_start(1, 1)
        idx_wait(0)
        gather_start(0)

        @pl.loop(0, n_chunks // 2)
        def _outer(j):
            for b in (0, 1):
                i = j * 2 + b
                nb = 1 - b
                gather_wait(b)

                @pl.when(i + 2 < n_chunks)
                def _():
                    idx_start(i + 2, b)

                @pl.when(i + 1 < n_chunks)
                def _():
                    idx_wait(nb)
                    @pl.when(i >= 1)
                    def _():
                        write_wait(nb)
                    gather_start(nb)

                write_start(i, b)

        write_wait(0)
        write_wait(1)

    return gather


def kernel(sequences_tokens, embedding_table):
    b, l = sequences_tokens.shape
    _, d = embedding_table.shape
    idx = sequences_tokens.reshape(b * l)
    out = _build_gather(b * l, d)(idx, embedding_table)
    return out.reshape(b, l, d)
